# Initial kernel scaffold; baseline (speedup 1.0000x reference)
import jax
import jax.numpy as jnp
from jax.experimental import pallas as pl


def kernel(x, table):
    raise NotImplementedError("write your pallas kernel here")


# speedup vs baseline: 1.6828x; 1.6828x over previous
import functools

import jax
import jax.numpy as jnp
from jax import lax
from jax.experimental import pallas as pl
from jax.experimental.pallas import tpu as pltpu
from jax.experimental.pallas import tpu_sc as plsc

CH = 128


@functools.partial(jax.jit, static_argnums=())
def kernel(x, table):
    R, C = x.shape
    V, D = table.shape
    B = R * C

    info = plsc.get_sparse_core_info()
    NC, NS = info.num_cores, info.num_subcores
    NW = NC * NS

    n_total_ch = B // CH
    n_ch = n_total_ch // NW
    assert n_ch * NW == n_total_ch and n_total_ch * CH == B

    xf = x.reshape(n_total_ch, CH).astype(jnp.int32)

    mesh = plsc.VectorSubcoreMesh(core_axis_name="c", subcore_axis_name="s")

    @functools.partial(
        pl.kernel,
        mesh=mesh,
        compiler_params=pltpu.CompilerParams(use_tc_tiling_on_sc=False),
        out_type=jax.ShapeDtypeStruct((B, D), jnp.float32),
        scratch_types=[
            pltpu.VMEM((n_ch, CH), jnp.int32),
            pltpu.VMEM((CH, D), jnp.float32),
            pltpu.SemaphoreType.DMA,
        ],
    )
    def gather_k(x_hbm, table_hbm, out_hbm, idx_v, rows_v, sem):
        wid = lax.axis_index("s") * NC + lax.axis_index("c")
        row0 = wid * n_ch
        pltpu.sync_copy(x_hbm.at[pl.ds(row0, n_ch)], idx_v)

        def chunk(c, carry):
            pltpu.async_copy(table_hbm.at[idx_v.at[c]], rows_v, sem).wait()
            pltpu.sync_copy(rows_v, out_hbm.at[pl.ds((row0 + c) * CH, CH)])
            return carry

        lax.fori_loop(0, n_ch, chunk, 0)

    out = gather_k(xf, table)
    return out.reshape(R, C, D)


# speedup vs baseline: 1.8732x; 1.1131x over previous
import functools

import jax
import jax.numpy as jnp
from jax import lax
from jax.experimental import pallas as pl
from jax.experimental.pallas import tpu as pltpu
from jax.experimental.pallas import tpu_sc as plsc

CH = 128
NBUF = 8


@functools.partial(jax.jit, static_argnums=())
def kernel(x, table):
    R, C = x.shape
    V, D = table.shape
    B = R * C

    info = plsc.get_sparse_core_info()
    NC, NS = info.num_cores, info.num_subcores
    NW = NC * NS

    n_total_ch = B // CH
    n_ch = n_total_ch // NW
    n_grp = n_ch // NBUF
    assert n_ch * NW == n_total_ch and n_total_ch * CH == B
    assert n_grp * NBUF == n_ch

    xf = x.reshape(n_total_ch, CH).astype(jnp.int32)

    mesh = plsc.VectorSubcoreMesh(core_axis_name="c", subcore_axis_name="s")

    @functools.partial(
        pl.kernel,
        mesh=mesh,
        compiler_params=pltpu.CompilerParams(use_tc_tiling_on_sc=False),
        out_type=jax.ShapeDtypeStruct((B, D), jnp.float32),
        scratch_types=[
            pltpu.VMEM((n_ch, CH), jnp.int32),
            pltpu.VMEM((NBUF, CH, D), jnp.float32),
            pltpu.SemaphoreType.DMA((NBUF,)),
            pltpu.SemaphoreType.DMA((NBUF,)),
        ],
    )
    def gather_k(x_hbm, table_hbm, out_hbm, idx_v, rows_v, gsem, wsem):
        wid = lax.axis_index("s") * NC + lax.axis_index("c")
        row0 = wid * n_ch
        pltpu.sync_copy(x_hbm.at[pl.ds(row0, n_ch)], idx_v)

        def group(g, carry):
            c0 = g * NBUF
            gathers = [
                pltpu.async_copy(
                    table_hbm.at[idx_v.at[c0 + b]], rows_v.at[b], gsem.at[b]
                )
                for b in range(NBUF)
            ]
            writes = []
            for b in range(NBUF):
                gathers[b].wait()
                writes.append(
                    pltpu.async_copy(
                        rows_v.at[b],
                        out_hbm.at[pl.ds((row0 + c0 + b) * CH, CH)],
                        wsem.at[b],
                    )
                )
            for w in writes:
                w.wait()
            return carry

        lax.fori_loop(0, n_grp, group, 0)

    out = gather_k(xf, table)
    return out.reshape(R, C, D)


# speedup vs baseline: 1.8737x; 1.0003x over previous
import functools

import jax
import jax.numpy as jnp
from jax import lax
from jax.experimental import pallas as pl
from jax.experimental.pallas import tpu as pltpu
from jax.experimental.pallas import tpu_sc as plsc

CH = 128
NBUF = 8


@functools.partial(jax.jit, static_argnums=())
def kernel(x, table):
    R, C = x.shape
    V, D = table.shape
    B = R * C

    info = plsc.get_sparse_core_info()
    NC, NS = info.num_cores, info.num_subcores
    NW = NC * NS

    n_total_ch = B // CH
    n_ch = n_total_ch // NW
    n_grp = n_ch // NBUF
    assert n_ch * NW == n_total_ch and n_total_ch * CH == B
    assert n_grp * NBUF == n_ch

    xf = x.reshape(n_total_ch, CH).astype(jnp.int32)

    mesh = plsc.VectorSubcoreMesh(core_axis_name="c", subcore_axis_name="s")

    @functools.partial(
        pl.kernel,
        mesh=mesh,
        compiler_params=pltpu.CompilerParams(use_tc_tiling_on_sc=False),
        out_type=jax.ShapeDtypeStruct((B, D), jnp.float32),
        scratch_types=[
            pltpu.VMEM((n_ch, CH), jnp.int32),
            pltpu.VMEM((NBUF, CH, D), jnp.float32),
            pltpu.SemaphoreType.DMA((NBUF,)),
            pltpu.SemaphoreType.DMA((NBUF,)),
        ],
    )
    def gather_k(x_hbm, table_hbm, out_hbm, idx_v, rows_v, gsem, wsem):
        wid = lax.axis_index("s") * NC + lax.axis_index("c")
        row0 = wid * n_ch
        pltpu.sync_copy(x_hbm.at[pl.ds(row0, n_ch)], idx_v)

        def group(g, carry):
            c0 = g * NBUF
            gathers = []
            for b in range(NBUF):
                @pl.when(g > 0)
                def _drain(b=b):
                    pltpu.make_async_copy(
                        rows_v.at[b],
                        out_hbm.at[pl.ds(row0 * CH, CH)],
                        wsem.at[b],
                    ).wait()

                gathers.append(
                    pltpu.async_copy(
                        table_hbm.at[idx_v.at[c0 + b]], rows_v.at[b], gsem.at[b]
                    )
                )
            for b in range(NBUF):
                gathers[b].wait()
                pltpu.async_copy(
                    rows_v.at[b],
                    out_hbm.at[pl.ds((row0 + c0 + b) * CH, CH)],
                    wsem.at[b],
                )
            return carry

        lax.fori_loop(0, n_grp, group, 0)
        for b in range(NBUF):
            pltpu.make_async_copy(
                rows_v.at[b],
                out_hbm.at[pl.ds(row0 * CH, CH)],
                wsem.at[b],
            ).wait()

    out = gather_k(xf, table)
    return out.reshape(R, C, D)
